# baseline (speedup 1.0000x reference)
import jax
import jax.numpy as jnp
from jax import lax
from jax.experimental import pallas as pl
from jax.experimental.pallas import tpu as pltpu
from jax.experimental.pallas import tpu_sc as plsc

E = 6_400_000
N = 100_000
W = 3 * N
NC = 2
NS = 16
L = 16
NW = NC * NS
CHUNK = 2000
GROUPS = CHUNK // L
NCH = E // CHUNK // NW
WPS = 18752
WPS_LAST = W - (NS - 1) * WPS


def _sc_body(dT_hbm, iT_hbm, zeros_hbm, out_hbm, *scr):
    ins = (scr[0:5], scr[5:10])
    vals = (scr[10:12], scr[12:14])
    idxs = (scr[14:16], scr[16:18])
    acc_s = scr[18]
    sem_in = (scr[19], scr[20])
    sem_st = (scr[21], scr[22])

    cid = lax.axis_index("c")
    sid = lax.axis_index("s")
    wid = cid * NS + sid

    r0 = sid * WPS

    @pl.when(sid < NS - 1)
    def _():
        pltpu.sync_copy(zeros_hbm.at[pl.ds(r0, WPS)], acc_s.at[pl.ds(r0, WPS)])

    @pl.when(sid == NS - 1)
    def _():
        pltpu.sync_copy(zeros_hbm.at[pl.ds((NS - 1) * WPS, WPS_LAST)],
                        acc_s.at[pl.ds((NS - 1) * WPS, WPS_LAST)])

    plsc.subcore_barrier()

    start = wid * NCH
    magic = jnp.full((L,), 0x5F3759DF, jnp.int32)

    def srcs(e0):
        return (dT_hbm.at[0, pl.ds(e0, CHUNK)],
                dT_hbm.at[1, pl.ds(e0, CHUNK)],
                dT_hbm.at[2, pl.ds(e0, CHUNK)],
                iT_hbm.at[0, pl.ds(e0, CHUNK)],
                iT_hbm.at[1, pl.ds(e0, CHUNK)])

    def fire_inputs(ci, k):
        e0 = (start + ci) * CHUNK
        for src, dst in zip(srcs(e0), ins[k]):
            pltpu.async_copy(src, dst, sem_in[k])

    def wait_inputs(k):
        for src, dst in zip(srcs(0), ins[k]):
            pltpu.make_async_copy(src, dst, sem_in[k]).wait()

    def fire_streams(k):
        pbuf, nbuf = vals[k]
        wibuf, wjbuf = idxs[k]
        pltpu.async_copy(pbuf, acc_s.at[wibuf], sem_st[k], add=True)
        pltpu.async_copy(nbuf, acc_s.at[wjbuf], sem_st[k], add=True)

    def wait_streams(k):
        pbuf, nbuf = vals[k]
        wibuf, wjbuf = idxs[k]
        pltpu.make_async_copy(pbuf, acc_s.at[wibuf], sem_st[k]).wait()
        pltpu.make_async_copy(nbuf, acc_s.at[wjbuf], sem_st[k]).wait()

    def compute(k):
        vx, vy, vz, vii, vjj = ins[k]
        pbuf, nbuf = vals[k]
        wibuf, wjbuf = idxs[k]

        def do_group(g, c_):
            o = g * L
            sl = pl.ds(o, L)
            sx = pl.ds(o, L)
            sy = pl.ds(o + CHUNK, L)
            sz = pl.ds(o + 2 * CHUNK, L)
            ax = vx[sl]
            ay = vy[sl]
            az = vz[sl]
            r2 = ax * ax + ay * ay + az * az + 1e-12
            bi = plsc.bitcast(r2, jnp.int32)
            y = plsc.bitcast(magic - lax.shift_right_logical(bi, 1), jnp.float32)
            xh = r2 * 0.5
            y = y * (1.5 - xh * y * y)
            y = y * (1.5 - xh * y * y)
            y = y * (1.5 - xh * y * y)
            s = 1.0 - y
            t = y - 1.0
            pbuf[sx] = s * ax
            pbuf[sy] = s * ay
            pbuf[sz] = s * az
            nbuf[sx] = t * ax
            nbuf[sy] = t * ay
            nbuf[sz] = t * az
            wa = vii[sl] * 3
            wb = vjj[sl] * 3
            wibuf[sx] = wa
            wibuf[sy] = wa + 1
            wibuf[sz] = wa + 2
            wjbuf[sx] = wb
            wjbuf[sy] = wb + 1
            wjbuf[sz] = wb + 2
            return c_

        lax.fori_loop(0, GROUPS, do_group, 0, unroll=False)

    fire_inputs(0, 0)

    def do_chunk(ci, carry):
        def phase(k):
            wait_inputs(k)

            @pl.when(ci < NCH - 1)
            def _():
                fire_inputs(ci + 1, 1 - k)

            @pl.when(ci >= 2)
            def _():
                wait_streams(k)

            compute(k)
            fire_streams(k)

        @pl.when(ci % 2 == 0)
        def _():
            phase(0)

        @pl.when(ci % 2 == 1)
        def _():
            phase(1)

        return carry

    lax.fori_loop(0, NCH, do_chunk, 0, unroll=False)
    wait_streams(0)
    wait_streams(1)

    plsc.subcore_barrier()

    @pl.when(sid < NS - 1)
    def _():
        pltpu.sync_copy(acc_s.at[pl.ds(r0, WPS)],
                        out_hbm.at[cid, pl.ds(r0, WPS)])

    @pl.when(sid == NS - 1)
    def _():
        pltpu.sync_copy(acc_s.at[pl.ds((NS - 1) * WPS, WPS_LAST)],
                        out_hbm.at[cid, pl.ds((NS - 1) * WPS, WPS_LAST)])


def _combine_body(a_ref, b_ref, o_ref):
    o_ref[...] = a_ref[...] + b_ref[...]


def kernel(edge_diff, edge_idx, n_atoms):
    del n_atoms
    dT = edge_diff.T
    iT = edge_idx.T
    zeros = jnp.zeros((W,), jnp.float32)
    mesh = plsc.VectorSubcoreMesh(core_axis_name="c", subcore_axis_name="s")
    fvec = pltpu.VMEM((CHUNK,), jnp.float32)
    ivec = pltpu.VMEM((CHUNK,), jnp.int32)
    f3vec = pltpu.VMEM((3 * CHUNK,), jnp.float32)
    i3vec = pltpu.VMEM((3 * CHUNK,), jnp.int32)
    in_set = [fvec, fvec, fvec, ivec, ivec]
    partials = pl.kernel(
        _sc_body,
        out_type=jax.ShapeDtypeStruct((NC, W), jnp.float32),
        compiler_params=pltpu.CompilerParams(
            needs_layout_passes=False, use_tc_tiling_on_sc=False),
        mesh=mesh,
        scratch_types=(
            in_set + in_set
            + [f3vec, f3vec] + [f3vec, f3vec]
            + [i3vec, i3vec] + [i3vec, i3vec]
            + [pltpu.VMEM_SHARED((W,), jnp.float32)]
            + [pltpu.SemaphoreType.DMA] * 4
        ),
    )(dT, iT, zeros)

    pa = partials[0].reshape(300, 1000)
    pb = partials[1].reshape(300, 1000)
    out = pl.pallas_call(
        _combine_body,
        out_shape=jax.ShapeDtypeStruct((300, 1000), jnp.float32),
    )(pa, pb)
    return out.reshape(N, 3)


# speedup vs baseline: 2.9230x; 2.9230x over previous
import jax
import jax.numpy as jnp
from jax import lax
from jax.experimental import pallas as pl
from jax.experimental.pallas import tpu as pltpu
from jax.experimental.pallas import tpu_sc as plsc

E = 6_400_000
N = 100_000
W = 3 * N
NC = 2
NS = 16
L = 16
NW = NC * NS
CHUNK = 2000
GROUPS = CHUNK // L
NCH = E // CHUNK // NW
WPS = 18752
WPS_LAST = W - (NS - 1) * WPS


def _sc_body(dx_hbm, dy_hbm, dz_hbm, ii_hbm, jj_hbm, zeros_hbm, out_hbm,
             *scr):
    ins = (scr[0:5], scr[5:10], scr[10:15])
    vals = (scr[15:17], scr[17:19], scr[19:21])
    idxs = (scr[21:23], scr[23:25], scr[25:27])
    acc_s = scr[27]
    sem_in = (scr[28], scr[29], scr[30])
    sem_st = (scr[31], scr[32], scr[33])

    cid = lax.axis_index("c")
    sid = lax.axis_index("s")
    wid = cid * NS + sid

    r0 = sid * WPS

    @pl.when(sid < NS - 1)
    def _():
        pltpu.sync_copy(zeros_hbm.at[pl.ds(r0, WPS)], acc_s.at[pl.ds(r0, WPS)])

    @pl.when(sid == NS - 1)
    def _():
        pltpu.sync_copy(zeros_hbm.at[pl.ds((NS - 1) * WPS, WPS_LAST)],
                        acc_s.at[pl.ds((NS - 1) * WPS, WPS_LAST)])

    plsc.subcore_barrier()

    start = wid * NCH
    magic = jnp.full((L,), 0x5F3759DF, jnp.int32)
    srcs = (dx_hbm, dy_hbm, dz_hbm, ii_hbm, jj_hbm)

    def fire_inputs(ci, k):
        e0 = (start + ci) * CHUNK
        for src, dst in zip(srcs, ins[k]):
            pltpu.async_copy(src.at[pl.ds(e0, CHUNK)], dst, sem_in[k])

    def wait_inputs(k):
        for src, dst in zip(srcs, ins[k]):
            pltpu.make_async_copy(src.at[pl.ds(0, CHUNK)], dst, sem_in[k]).wait()

    def fire_streams(k):
        pbuf, nbuf = vals[k]
        wibuf, wjbuf = idxs[k]
        pltpu.async_copy(pbuf, acc_s.at[wibuf], sem_st[k], add=True)
        pltpu.async_copy(nbuf, acc_s.at[wjbuf], sem_st[k], add=True)

    def wait_streams(k):
        pbuf, nbuf = vals[k]
        wibuf, wjbuf = idxs[k]
        pltpu.make_async_copy(pbuf, acc_s.at[wibuf], sem_st[k]).wait()
        pltpu.make_async_copy(nbuf, acc_s.at[wjbuf], sem_st[k]).wait()

    def compute(k):
        vx, vy, vz, vii, vjj = ins[k]
        pbuf, nbuf = vals[k]
        wibuf, wjbuf = idxs[k]

        def do_group(g, c_):
            o = g * L
            sl = pl.ds(o, L)
            sx = pl.ds(o, L)
            sy = pl.ds(o + CHUNK, L)
            sz = pl.ds(o + 2 * CHUNK, L)
            ax = vx[sl]
            ay = vy[sl]
            az = vz[sl]
            r2 = ax * ax + ay * ay + az * az + 1e-12
            bi = plsc.bitcast(r2, jnp.int32)
            y = plsc.bitcast(magic - lax.shift_right_logical(bi, 1), jnp.float32)
            xh = r2 * 0.5
            y = y * (1.5 - xh * y * y)
            y = y * (1.5 - xh * y * y)
            y = y * (1.5 - xh * y * y)
            s = 1.0 - y
            t = y - 1.0
            pbuf[sx] = s * ax
            pbuf[sy] = s * ay
            pbuf[sz] = s * az
            nbuf[sx] = t * ax
            nbuf[sy] = t * ay
            nbuf[sz] = t * az
            wa = vii[sl] * 3
            wb = vjj[sl] * 3
            wibuf[sx] = wa
            wibuf[sy] = wa + 1
            wibuf[sz] = wa + 2
            wjbuf[sx] = wb
            wjbuf[sy] = wb + 1
            wjbuf[sz] = wb + 2
            return c_

        lax.fori_loop(0, GROUPS, do_group, 0, unroll=False)

    fire_inputs(0, 0)

    def do_chunk(ci, carry):
        def phase(k):
            wait_inputs(k)

            @pl.when(ci < NCH - 1)
            def _():
                fire_inputs(ci + 1, (k + 1) % 3)

            @pl.when(ci >= 3)
            def _():
                wait_streams(k)

            compute(k)
            fire_streams(k)

        @pl.when(ci % 3 == 0)
        def _():
            phase(0)

        @pl.when(ci % 3 == 1)
        def _():
            phase(1)

        @pl.when(ci % 3 == 2)
        def _():
            phase(2)

        return carry

    lax.fori_loop(0, NCH, do_chunk, 0, unroll=False)
    wait_streams(0)
    wait_streams(1)
    wait_streams(2)

    plsc.subcore_barrier()

    @pl.when(sid < NS - 1)
    def _():
        pltpu.sync_copy(acc_s.at[pl.ds(r0, WPS)],
                        out_hbm.at[cid, pl.ds(r0, WPS)])

    @pl.when(sid == NS - 1)
    def _():
        pltpu.sync_copy(acc_s.at[pl.ds((NS - 1) * WPS, WPS_LAST)],
                        out_hbm.at[cid, pl.ds((NS - 1) * WPS, WPS_LAST)])


def _combine_body(a_ref, b_ref, o_ref):
    o_ref[...] = a_ref[...] + b_ref[...]


def kernel(edge_diff, edge_idx, n_atoms):
    del n_atoms
    dx = edge_diff[:, 0]
    dy = edge_diff[:, 1]
    dz = edge_diff[:, 2]
    ii = edge_idx[:, 0]
    jj = edge_idx[:, 1]
    zeros = jnp.zeros((W,), jnp.float32)
    mesh = plsc.VectorSubcoreMesh(core_axis_name="c", subcore_axis_name="s")
    fvec = pltpu.VMEM((CHUNK,), jnp.float32)
    ivec = pltpu.VMEM((CHUNK,), jnp.int32)
    f3vec = pltpu.VMEM((3 * CHUNK,), jnp.float32)
    i3vec = pltpu.VMEM((3 * CHUNK,), jnp.int32)
    in_set = [fvec, fvec, fvec, ivec, ivec]
    partials = pl.kernel(
        _sc_body,
        out_type=jax.ShapeDtypeStruct((NC, W), jnp.float32),
        compiler_params=pltpu.CompilerParams(
            needs_layout_passes=False, use_tc_tiling_on_sc=False),
        mesh=mesh,
        scratch_types=(
            in_set + in_set + in_set
            + [f3vec, f3vec] * 3
            + [i3vec, i3vec] * 3
            + [pltpu.VMEM_SHARED((W,), jnp.float32)]
            + [pltpu.SemaphoreType.DMA] * 6
        ),
    )(dx, dy, dz, ii, jj, zeros)

    pa = partials[0].reshape(300, 1000)
    pb = partials[1].reshape(300, 1000)
    out = pl.pallas_call(
        _combine_body,
        out_shape=jax.ShapeDtypeStruct((300, 1000), jnp.float32),
    )(pa, pb)
    return out.reshape(N, 3)
